# Initial kernel scaffold; baseline (speedup 1.0000x reference)
import jax
import jax.numpy as jnp
from jax.experimental import pallas as pl


def kernel(D):
    raise NotImplementedError("write your pallas kernel here")


# speedup vs baseline: 3690.1801x; 3690.1801x over previous
import functools

import jax
import jax.numpy as jnp
from jax import lax
from jax.experimental import pallas as pl
from jax.experimental.pallas import tpu as pltpu
from jax.experimental.pallas import tpu_sc as plsc

N = 64
L = 16
NV = N // L
NC, NS = 2, 16
PAD = 16
BW = PAD + N
PROW = 80
INF = float("inf")

_mesh = plsc.VectorSubcoreMesh(core_axis_name="c", subcore_axis_name="s")


@functools.partial(
    pl.kernel,
    out_type=[
        jax.ShapeDtypeStruct((L,), jnp.float32),
        jax.ShapeDtypeStruct((N, N), jnp.float32),
    ],
    mesh=_mesh,
    compiler_params=pltpu.CompilerParams(needs_layout_passes=False),
    scratch_types=[
        pltpu.VMEM((N * N,), jnp.float32),
        pltpu.VMEM((BW,), jnp.float32),
        pltpu.VMEM((BW,), jnp.float32),
        pltpu.VMEM((3 * PROW,), jnp.float32),
        pltpu.VMEM((N,), jnp.float32),
        pltpu.VMEM((N,), jnp.float32),
    ],
)
def _sdtw_sc(d_hbm, cost_hbm, grad_hbm, dmat, bufA, bufB, pflat, grow0, grow1):
    wid = lax.axis_index("s") * NC + lax.axis_index("c")
    iota = lax.iota(jnp.int32, L)
    jv = [iota + (L * c) for c in range(NV)]
    inf_vec = jnp.full((L,), INF, jnp.float32)
    zero_vec = jnp.zeros((L,), jnp.float32)

    pltpu.sync_copy(d_hbm, dmat)

    r0 = wid * 2
    for r in range(3):
        ri = r0 + r
        row_ok = ri <= N - 1
        rclamp = jnp.minimum(ri, N - 1)
        for c in range(NV):
            g = plsc.load_gather(dmat, [jv[c] + rclamp * N])
            v = jnp.exp(-jnp.exp(-g))
            v = jnp.where(row_ok, v, 0.0)
            pflat[pl.ds(r * PROW + c * L, L)] = v
        pflat[pl.ds(r * PROW + N, L)] = zero_vec
    for r, grow in ((0, grow0), (1, grow1)):
        for c in range(NV):
            a = pflat[pl.ds(r * PROW + c * L, L)]
            ash = plsc.load_gather(pflat, [iota + (r * PROW + c * L + 1)])
            b = pflat[pl.ds((r + 1) * PROW + c * L, L)]
            bsh = plsc.load_gather(pflat, [iota + ((r + 1) * PROW + c * L + 1)])
            grow[pl.ds(c * L, L)] = ash + b + bsh - a
    pltpu.sync_copy(grow0, grad_hbm.at[r0])
    pltpu.sync_copy(grow1, grad_hbm.at[r0 + 1])

    @pl.when(wid == 0)
    def _dp():
        shidx = [iota + (PAD - 1 + L * c) for c in range(NV)]

        def dp_step(d, prevbuf, curbuf):
            pv, ps, p2, dv = [], [], [], []
            for c in range(NV):
                pv.append(prevbuf[pl.ds(PAD + L * c, L)])
                ps.append(plsc.load_gather(prevbuf, [shidx[c]]))
                p2.append(plsc.load_gather(curbuf, [shidx[c]]))
                row = d - jv[c]
                fidx = jnp.clip(row * N + jv[c], 0, N * N - 1)
                g = plsc.load_gather(dmat, [fidx])
                valid = (row >= 0) & (row <= N - 1)
                dv.append(jnp.where(valid, jnp.exp(-g), INF))
            for c in range(NV):
                cur = jnp.minimum(jnp.minimum(pv[c], ps[c]), p2[c]) + dv[c]
                curbuf[pl.ds(PAD + L * c, L)] = cur

        for off in range(0, BW, L):
            bufA[pl.ds(off, L)] = inf_vec
            bufB[pl.ds(off, L)] = inf_vec
        bufB[pl.ds(0, L)] = jnp.where(iota == L - 1, 0.0, INF)
        dp_step(jnp.int32(0), bufB, bufA)
        bufB[pl.ds(0, L)] = inf_vec

        def body(t, carry):
            dp_step(2 * t + 1, bufA, bufB)
            dp_step(2 * t + 2, bufB, bufA)
            return carry

        lax.fori_loop(0, (2 * N - 2) // 2, body, jnp.int32(0))
        pltpu.sync_copy(bufA.at[pl.ds(BW - L, L)], cost_hbm)


def kernel(D):
    cost16, grad = _sdtw_sc(D.reshape(N * N))
    return cost16[L - 1], grad
